# baseline (speedup 1.0000x reference)
import jax
import jax.numpy as jnp
from jax import lax
from jax.experimental import pallas as pl
from jax.experimental.pallas import tpu as pltpu
from jax.experimental.pallas import tpu_sc as plsc

N_NODES = 100000
N_EDGES = 1600000
N_CELLS = 200000

NC = 2
NS = 16
NW = NC * NS

CHUNK = 2000
EPT = N_EDGES // NW
E_CHUNKS = EPT // CHUNK
C_CHUNKS = N_CELLS // CHUNK
N_CHUNKS = N_NODES // CHUNK
LANES = 16
GROUPS = CHUNK // LANES

_mesh = plsc.VectorSubcoreMesh(core_axis_name="c", subcore_axis_name="s")
_params = pltpu.CompilerParams(use_tc_tiling_on_sc=False)


def _iota_fill(ref, base3, mul=3):
    iota = lax.iota(jnp.int32, LANES)

    def _f(i, carry):
        p = (base3 + i * LANES + iota) * mul
        ref[pl.ds(i * LANES, LANES)] = p
        return carry
    lax.fori_loop(0, GROUPS, _f, None)


def _inc(ref):
    def _f(i, carry):
        sl = pl.ds(i * LANES, LANES)
        ref[sl] = ref[sl] + 1
        return carry
    lax.fori_loop(0, GROUPS, _f, None)


def _stage1_body(uvpf_hbm, send_hbm, recv_hbm,
                 f0_hbm, f1_hbm, f2_hbm, zeros_hbm,
                 u0_out, v0_out, p0_out, n0_out,
                 u1_out, v1_out, p1_out, n1_out,
                 cellf_out,
                 accu_sp, accv_sp, accp_sp, accn_sp,
                 su_v, sv_v, sp_v, ones_v, g_v, a0_v, a1_v, a2_v,
                 sidx_v, ridx_v, f_v, ix_v):
    cid = lax.axis_index("c")
    sid = lax.axis_index("s")
    wid = sid * NC + cid

    for t in range(4):
        ch = sid + NS * t
        @pl.when(ch < N_CHUNKS)
        def _():
            sl = pl.ds(ch * CHUNK, CHUNK)
            pltpu.sync_copy(zeros_hbm, accu_sp.at[sl])
            pltpu.sync_copy(zeros_hbm, accv_sp.at[sl])
            pltpu.sync_copy(zeros_hbm, accp_sp.at[sl])
            pltpu.sync_copy(zeros_hbm, accn_sp.at[sl])

    def _fill_ones(i, carry):
        ones_v[pl.ds(i * LANES, LANES)] = jnp.full((LANES,), 1.0, jnp.float32)
        return carry
    lax.fori_loop(0, GROUPS, _fill_ones, None)

    plsc.subcore_barrier()

    for j in range(E_CHUNKS):
        base = wid * EPT + j * CHUNK
        sl = pl.ds(base, CHUNK)
        _iota_fill(ix_v, base)
        pltpu.sync_copy(uvpf_hbm.at[ix_v], su_v)
        _inc(ix_v)
        pltpu.sync_copy(uvpf_hbm.at[ix_v], sv_v)
        _inc(ix_v)
        pltpu.sync_copy(uvpf_hbm.at[ix_v], sp_v)
        pltpu.sync_copy(send_hbm.at[sl], sidx_v)
        pltpu.sync_copy(recv_hbm.at[sl], ridx_v)
        for ix in (sidx_v, ridx_v):
            pltpu.sync_copy(su_v, accu_sp.at[ix], add=True)
            pltpu.sync_copy(sv_v, accv_sp.at[ix], add=True)
            pltpu.sync_copy(sp_v, accp_sp.at[ix], add=True)
            pltpu.sync_copy(ones_v, accn_sp.at[ix], add=True)

    for t in range(4):
        ch = wid + NW * t
        @pl.when(ch < C_CHUNKS)
        def _():
            base = ch * CHUNK
            accs = (a0_v, a1_v, a2_v)
            for k, fk in enumerate((f0_hbm, f1_hbm, f2_hbm)):
                pltpu.sync_copy(fk.at[pl.ds(base, CHUNK)], f_v)

                def _mk_idx(i, c):
                    sl = pl.ds(i * LANES, LANES)
                    ix_v[sl] = f_v[sl] * 3
                    return c
                lax.fori_loop(0, GROUPS, _mk_idx, None)

                for m in range(3):
                    if k == 0:
                        pltpu.sync_copy(uvpf_hbm.at[ix_v], accs[m])
                    else:
                        pltpu.sync_copy(uvpf_hbm.at[ix_v], g_v)

                        def _acc(i, c, am=accs[m]):
                            sl = pl.ds(i * LANES, LANES)
                            am[sl] = am[sl] + g_v[sl]
                            return c
                        lax.fori_loop(0, GROUPS, _acc, None)
                    if m < 2:
                        _inc(ix_v)

            def _scale(i, c):
                sl = pl.ds(i * LANES, LANES)
                a0_v[sl] = a0_v[sl] / 3.0
                a1_v[sl] = a1_v[sl] / 3.0
                a2_v[sl] = a2_v[sl] / 3.0
                return c
            lax.fori_loop(0, GROUPS, _scale, None)

            _iota_fill(ix_v, base)
            pltpu.sync_copy(a0_v, cellf_out.at[ix_v])
            _inc(ix_v)
            pltpu.sync_copy(a1_v, cellf_out.at[ix_v])
            _inc(ix_v)
            pltpu.sync_copy(a2_v, cellf_out.at[ix_v])

    plsc.subcore_barrier()
    for t in range(4):
        ch = sid + NS * t
        @pl.when(ch < N_CHUNKS)
        def _():
            sl = pl.ds(ch * CHUNK, CHUNK)
            @pl.when(cid == 0)
            def _():
                pltpu.sync_copy(accu_sp.at[sl], u0_out.at[sl])
                pltpu.sync_copy(accv_sp.at[sl], v0_out.at[sl])
                pltpu.sync_copy(accp_sp.at[sl], p0_out.at[sl])
                pltpu.sync_copy(accn_sp.at[sl], n0_out.at[sl])
            @pl.when(cid == 1)
            def _():
                pltpu.sync_copy(accu_sp.at[sl], u1_out.at[sl])
                pltpu.sync_copy(accv_sp.at[sl], v1_out.at[sl])
                pltpu.sync_copy(accp_sp.at[sl], p1_out.at[sl])
                pltpu.sync_copy(accn_sp.at[sl], n1_out.at[sl])


_N1 = jax.ShapeDtypeStruct((N_NODES,), jnp.float32)

_stage1 = pl.kernel(
    _stage1_body,
    out_type=(_N1, _N1, _N1, _N1, _N1, _N1, _N1, _N1,
              jax.ShapeDtypeStruct((N_CELLS * 3,), jnp.float32)),
    mesh=_mesh,
    compiler_params=_params,
    scratch_types=(
        [pltpu.VMEM_SHARED((N_NODES,), jnp.float32)] * 4
        + [pltpu.VMEM((CHUNK,), jnp.float32)] * 8
        + [pltpu.VMEM((CHUNK,), jnp.int32)] * 4
    ),
)


def _stage2_body(u0_hbm, v0_hbm, p0_hbm, n0_hbm, u1_hbm, v1_hbm, p1_hbm, n1_hbm,
                 nodef_out,
                 a_v, b_v, c0_v, c1_v, den_v, o_v, ix_v):
    cid = lax.axis_index("c")
    sid = lax.axis_index("s")
    wid = sid * NC + cid

    for t in range(2):
        ch = wid + NW * t
        @pl.when(ch < N_CHUNKS)
        def _():
            sl = pl.ds(ch * CHUNK, CHUNK)
            pltpu.sync_copy(n0_hbm.at[sl], c0_v)
            pltpu.sync_copy(n1_hbm.at[sl], c1_v)

            def _den(i, c):
                s = pl.ds(i * LANES, LANES)
                den_v[s] = jnp.maximum(c0_v[s] + c1_v[s], 1.0)
                return c
            lax.fori_loop(0, GROUPS, _den, None)

            _iota_fill(ix_v, ch * CHUNK)
            for (x0, x1) in ((u0_hbm, u1_hbm), (v0_hbm, v1_hbm),
                             (p0_hbm, p1_hbm)):
                pltpu.sync_copy(x0.at[sl], a_v)
                pltpu.sync_copy(x1.at[sl], b_v)

                def _norm(i, c):
                    s = pl.ds(i * LANES, LANES)
                    o_v[s] = (a_v[s] + b_v[s]) / den_v[s]
                    return c
                lax.fori_loop(0, GROUPS, _norm, None)
                pltpu.sync_copy(o_v, nodef_out.at[ix_v])
                if x0 is not p0_hbm:
                    _inc(ix_v)


_stage2 = pl.kernel(
    _stage2_body,
    out_type=jax.ShapeDtypeStruct((N_NODES * 3,), jnp.float32),
    mesh=_mesh,
    compiler_params=_params,
    scratch_types=([pltpu.VMEM((CHUNK,), jnp.float32)] * 6
                   + [pltpu.VMEM((CHUNK,), jnp.int32)]),
)


@jax.jit
def kernel(predicted_edge_uvp, edge_index, face):
    senders = edge_index[0]
    receivers = edge_index[1]
    f0, f1, f2 = face[0], face[1], face[2]
    uvp_flat = predicted_edge_uvp.reshape(N_EDGES * 3)
    zeros = jnp.zeros((CHUNK,), jnp.float32)
    u0, v0, p0, n0, u1, v1, p1, n1, cellf = _stage1(
        uvp_flat, senders, receivers, f0, f1, f2, zeros)
    nodef = _stage2(u0, v0, p0, n0, u1, v1, p1, n1)
    return nodef.reshape(N_NODES, 3), cellf.reshape(N_CELLS, 3)


# speedup vs baseline: 3.9935x; 3.9935x over previous
import jax
import jax.numpy as jnp
from jax import lax
from jax.experimental import pallas as pl
from jax.experimental.pallas import tpu as pltpu
from jax.experimental.pallas import tpu_sc as plsc

N_NODES = 100000
N_EDGES = 1600000
N_CELLS = 200000

NC = 2
NS = 16
NW = NC * NS

CHUNK = 2000
EPT = N_EDGES // NW
E_CHUNKS = EPT // CHUNK
C_CHUNKS = N_CELLS // CHUNK
N_CHUNKS = N_NODES // CHUNK
LANES = 16
GROUPS = CHUNK // LANES

COL_BLK = 12800

_mesh = plsc.VectorSubcoreMesh(core_axis_name="c", subcore_axis_name="s")
_params = pltpu.CompilerParams(use_tc_tiling_on_sc=False)


def _cols_body(x_ref, u_ref, v_ref, p_ref):
    i = pl.program_id(0)
    x = x_ref[...]
    sl = pl.ds(i * COL_BLK, COL_BLK)
    u_ref[sl] = x[:, 0]
    v_ref[sl] = x[:, 1]
    p_ref[sl] = x[:, 2]


_split_cols = pl.pallas_call(
    _cols_body,
    grid=(N_EDGES // COL_BLK,),
    in_specs=[pl.BlockSpec((COL_BLK, 3), lambda i: (i, 0))],
    out_specs=[pl.BlockSpec((N_EDGES,), lambda i: (0,))] * 3,
    out_shape=[jax.ShapeDtypeStruct((N_EDGES,), jnp.float32)] * 3,
)


def _stage1_body(u_hbm, v_hbm, p_hbm, send_hbm, recv_hbm,
                 f0_hbm, f1_hbm, f2_hbm, zeros_hbm,
                 u0_out, v0_out, p0_out, n0_out,
                 u1_out, v1_out, p1_out, n1_out,
                 cell0_out, cell1_out, cell2_out,
                 accu_sp, accv_sp, accp_sp, accn_sp,
                 su_v, sv_v, sp_v, ones_v, g_v, a0_v, a1_v, a2_v,
                 sidx_v, ridx_v, f_v):
    cid = lax.axis_index("c")
    sid = lax.axis_index("s")
    wid = sid * NC + cid

    for t in range(4):
        ch = sid + NS * t
        @pl.when(ch < N_CHUNKS)
        def _():
            sl = pl.ds(ch * CHUNK, CHUNK)
            pltpu.sync_copy(zeros_hbm, accu_sp.at[sl])
            pltpu.sync_copy(zeros_hbm, accv_sp.at[sl])
            pltpu.sync_copy(zeros_hbm, accp_sp.at[sl])
            pltpu.sync_copy(zeros_hbm, accn_sp.at[sl])

    def _fill_ones(i, carry):
        ones_v[pl.ds(i * LANES, LANES)] = jnp.full((LANES,), 1.0, jnp.float32)
        return carry
    lax.fori_loop(0, GROUPS, _fill_ones, None)

    plsc.subcore_barrier()

    for j in range(E_CHUNKS):
        base = wid * EPT + j * CHUNK
        sl = pl.ds(base, CHUNK)
        pltpu.sync_copy(u_hbm.at[sl], su_v)
        pltpu.sync_copy(v_hbm.at[sl], sv_v)
        pltpu.sync_copy(p_hbm.at[sl], sp_v)
        pltpu.sync_copy(send_hbm.at[sl], sidx_v)
        pltpu.sync_copy(recv_hbm.at[sl], ridx_v)
        for ix in (sidx_v, ridx_v):
            pltpu.sync_copy(su_v, accu_sp.at[ix], add=True)
            pltpu.sync_copy(sv_v, accv_sp.at[ix], add=True)
            pltpu.sync_copy(sp_v, accp_sp.at[ix], add=True)
            pltpu.sync_copy(ones_v, accn_sp.at[ix], add=True)

    cols = (u_hbm, v_hbm, p_hbm)
    for t in range(4):
        ch = wid + NW * t
        @pl.when(ch < C_CHUNKS)
        def _():
            base = ch * CHUNK
            accs = (a0_v, a1_v, a2_v)
            for k, fk in enumerate((f0_hbm, f1_hbm, f2_hbm)):
                pltpu.sync_copy(fk.at[pl.ds(base, CHUNK)], f_v)
                for m in range(3):
                    if k == 0:
                        pltpu.sync_copy(cols[m].at[f_v], accs[m])
                    else:
                        pltpu.sync_copy(cols[m].at[f_v], g_v)

                        def _acc(i, c, am=accs[m]):
                            sl = pl.ds(i * LANES, LANES)
                            am[sl] = am[sl] + g_v[sl]
                            return c
                        lax.fori_loop(0, GROUPS, _acc, None)

            def _scale(i, c):
                sl = pl.ds(i * LANES, LANES)
                a0_v[sl] = a0_v[sl] / 3.0
                a1_v[sl] = a1_v[sl] / 3.0
                a2_v[sl] = a2_v[sl] / 3.0
                return c
            lax.fori_loop(0, GROUPS, _scale, None)

            pltpu.sync_copy(a0_v, cell0_out.at[pl.ds(base, CHUNK)])
            pltpu.sync_copy(a1_v, cell1_out.at[pl.ds(base, CHUNK)])
            pltpu.sync_copy(a2_v, cell2_out.at[pl.ds(base, CHUNK)])

    plsc.subcore_barrier()
    for t in range(4):
        ch = sid + NS * t
        @pl.when(ch < N_CHUNKS)
        def _():
            sl = pl.ds(ch * CHUNK, CHUNK)
            @pl.when(cid == 0)
            def _():
                pltpu.sync_copy(accu_sp.at[sl], u0_out.at[sl])
                pltpu.sync_copy(accv_sp.at[sl], v0_out.at[sl])
                pltpu.sync_copy(accp_sp.at[sl], p0_out.at[sl])
                pltpu.sync_copy(accn_sp.at[sl], n0_out.at[sl])
            @pl.when(cid == 1)
            def _():
                pltpu.sync_copy(accu_sp.at[sl], u1_out.at[sl])
                pltpu.sync_copy(accv_sp.at[sl], v1_out.at[sl])
                pltpu.sync_copy(accp_sp.at[sl], p1_out.at[sl])
                pltpu.sync_copy(accn_sp.at[sl], n1_out.at[sl])


_N1 = jax.ShapeDtypeStruct((N_NODES,), jnp.float32)
_C1 = jax.ShapeDtypeStruct((N_CELLS,), jnp.float32)

_stage1 = pl.kernel(
    _stage1_body,
    out_type=(_N1, _N1, _N1, _N1, _N1, _N1, _N1, _N1, _C1, _C1, _C1),
    mesh=_mesh,
    compiler_params=_params,
    scratch_types=(
        [pltpu.VMEM_SHARED((N_NODES,), jnp.float32)] * 4
        + [pltpu.VMEM((CHUNK,), jnp.float32)] * 8
        + [pltpu.VMEM((CHUNK,), jnp.int32)] * 3
    ),
)


def _stage2_body(u0_hbm, v0_hbm, p0_hbm, n0_hbm, u1_hbm, v1_hbm, p1_hbm, n1_hbm,
                 nu_out, nv_out, np_out,
                 a_v, b_v, c0_v, c1_v, den_v, o_v):
    cid = lax.axis_index("c")
    sid = lax.axis_index("s")
    wid = sid * NC + cid

    for t in range(2):
        ch = wid + NW * t
        @pl.when(ch < N_CHUNKS)
        def _():
            sl = pl.ds(ch * CHUNK, CHUNK)
            pltpu.sync_copy(n0_hbm.at[sl], c0_v)
            pltpu.sync_copy(n1_hbm.at[sl], c1_v)

            def _den(i, c):
                s = pl.ds(i * LANES, LANES)
                den_v[s] = jnp.maximum(c0_v[s] + c1_v[s], 1.0)
                return c
            lax.fori_loop(0, GROUPS, _den, None)

            for (x0, x1, out) in ((u0_hbm, u1_hbm, nu_out),
                                  (v0_hbm, v1_hbm, nv_out),
                                  (p0_hbm, p1_hbm, np_out)):
                pltpu.sync_copy(x0.at[sl], a_v)
                pltpu.sync_copy(x1.at[sl], b_v)

                def _norm(i, c):
                    s = pl.ds(i * LANES, LANES)
                    o_v[s] = (a_v[s] + b_v[s]) / den_v[s]
                    return c
                lax.fori_loop(0, GROUPS, _norm, None)
                pltpu.sync_copy(o_v, out.at[sl])


_stage2 = pl.kernel(
    _stage2_body,
    out_type=(_N1, _N1, _N1),
    mesh=_mesh,
    compiler_params=_params,
    scratch_types=[pltpu.VMEM((CHUNK,), jnp.float32)] * 6,
)


@jax.jit
def kernel(predicted_edge_uvp, edge_index, face):
    senders = edge_index[0]
    receivers = edge_index[1]
    f0, f1, f2 = face[0], face[1], face[2]
    u_col, v_col, p_col = _split_cols(predicted_edge_uvp)
    zeros = jnp.zeros((CHUNK,), jnp.float32)
    u0, v0, p0, n0, u1, v1, p1, n1, c0, c1, c2 = _stage1(
        u_col, v_col, p_col, senders, receivers, f0, f1, f2, zeros)
    nu, nv, np_ = _stage2(u0, v0, p0, n0, u1, v1, p1, n1)
    node_uvp = jnp.stack([nu, nv, np_], axis=1)
    cell_uvp = jnp.stack([c0, c1, c2], axis=1)
    return node_uvp, cell_uvp


# speedup vs baseline: 15.7747x; 3.9501x over previous
import jax
import jax.numpy as jnp
from jax import lax
from jax.experimental import pallas as pl
from jax.experimental.pallas import tpu as pltpu
from jax.experimental.pallas import tpu_sc as plsc

N_NODES = 100000
N_EDGES = 1600000
N_CELLS = 200000

NC = 2
NS = 16
NW = NC * NS

CHUNK = 2000
EPT = N_EDGES // NW
E_CHUNKS = EPT // CHUNK
C_CHUNKS = N_CELLS // CHUNK
N_CHUNKS = N_NODES // CHUNK
LANES = 16
GROUPS = CHUNK // LANES

COL_BLK = 12800

_mesh = plsc.VectorSubcoreMesh(core_axis_name="c", subcore_axis_name="s")
_params = pltpu.CompilerParams(use_tc_tiling_on_sc=False)


def _edge_body(ut_ref, ei_ref, u_ref, v_ref, p_ref, s_ref, r_ref):
    i = pl.program_id(0)
    sl = pl.ds(i * COL_BLK, COL_BLK)
    ut = ut_ref[...]
    u_ref[sl] = ut[0, :]
    v_ref[sl] = ut[1, :]
    p_ref[sl] = ut[2, :]
    ei = ei_ref[...]
    s_ref[sl] = ei[0, :]
    r_ref[sl] = ei[1, :]


_split_edges = pl.pallas_call(
    _edge_body,
    grid=(N_EDGES // COL_BLK,),
    in_specs=[pl.BlockSpec((3, COL_BLK), lambda i: (0, i)),
              pl.BlockSpec((2, COL_BLK), lambda i: (0, i))],
    out_specs=[pl.BlockSpec((N_EDGES,), lambda i: (0,))] * 5,
    out_shape=[jax.ShapeDtypeStruct((N_EDGES,), jnp.float32)] * 3
    + [jax.ShapeDtypeStruct((N_EDGES,), jnp.int32)] * 2,
)

def _face_body(f_ref, f0_ref, f1_ref, f2_ref):
    f = f_ref[...]
    f0_ref[...] = f[0, :]
    f1_ref[...] = f[1, :]
    f2_ref[...] = f[2, :]


_split_faces = pl.pallas_call(
    _face_body,
    out_shape=[jax.ShapeDtypeStruct((N_CELLS,), jnp.int32)] * 3,
)


def _stage1_body(u_hbm, v_hbm, p_hbm, send_hbm, recv_hbm,
                 f0_hbm, f1_hbm, f2_hbm, zeros_hbm,
                 u0_out, v0_out, p0_out, n0_out,
                 u1_out, v1_out, p1_out, n1_out,
                 cell0_out, cell1_out, cell2_out,
                 accu_sp, accv_sp, accp_sp, accn_sp,
                 su_v, sv_v, sp_v, ones_v, g_v, a0_v, a1_v, a2_v,
                 sidx_v, ridx_v, f_v):
    cid = lax.axis_index("c")
    sid = lax.axis_index("s")
    wid = sid * NC + cid

    for t in range(4):
        ch = sid + NS * t
        @pl.when(ch < N_CHUNKS)
        def _():
            sl = pl.ds(ch * CHUNK, CHUNK)
            pltpu.sync_copy(zeros_hbm, accu_sp.at[sl])
            pltpu.sync_copy(zeros_hbm, accv_sp.at[sl])
            pltpu.sync_copy(zeros_hbm, accp_sp.at[sl])
            pltpu.sync_copy(zeros_hbm, accn_sp.at[sl])

    def _fill_ones(i, carry):
        ones_v[pl.ds(i * LANES, LANES)] = jnp.full((LANES,), 1.0, jnp.float32)
        return carry
    lax.fori_loop(0, GROUPS, _fill_ones, None)

    plsc.subcore_barrier()

    for j in range(E_CHUNKS):
        base = wid * EPT + j * CHUNK
        sl = pl.ds(base, CHUNK)
        pltpu.sync_copy(u_hbm.at[sl], su_v)
        pltpu.sync_copy(v_hbm.at[sl], sv_v)
        pltpu.sync_copy(p_hbm.at[sl], sp_v)
        pltpu.sync_copy(send_hbm.at[sl], sidx_v)
        pltpu.sync_copy(recv_hbm.at[sl], ridx_v)
        for ix in (sidx_v, ridx_v):
            pltpu.sync_copy(su_v, accu_sp.at[ix], add=True)
            pltpu.sync_copy(sv_v, accv_sp.at[ix], add=True)
            pltpu.sync_copy(sp_v, accp_sp.at[ix], add=True)
            pltpu.sync_copy(ones_v, accn_sp.at[ix], add=True)

    cols = (u_hbm, v_hbm, p_hbm)
    for t in range(4):
        ch = wid + NW * t
        @pl.when(ch < C_CHUNKS)
        def _():
            base = ch * CHUNK
            accs = (a0_v, a1_v, a2_v)
            for k, fk in enumerate((f0_hbm, f1_hbm, f2_hbm)):
                pltpu.sync_copy(fk.at[pl.ds(base, CHUNK)], f_v)
                for m in range(3):
                    if k == 0:
                        pltpu.sync_copy(cols[m].at[f_v], accs[m])
                    else:
                        pltpu.sync_copy(cols[m].at[f_v], g_v)

                        def _acc(i, c, am=accs[m]):
                            sl = pl.ds(i * LANES, LANES)
                            am[sl] = am[sl] + g_v[sl]
                            return c
                        lax.fori_loop(0, GROUPS, _acc, None)

            def _scale(i, c):
                sl = pl.ds(i * LANES, LANES)
                a0_v[sl] = a0_v[sl] / 3.0
                a1_v[sl] = a1_v[sl] / 3.0
                a2_v[sl] = a2_v[sl] / 3.0
                return c
            lax.fori_loop(0, GROUPS, _scale, None)

            pltpu.sync_copy(a0_v, cell0_out.at[pl.ds(base, CHUNK)])
            pltpu.sync_copy(a1_v, cell1_out.at[pl.ds(base, CHUNK)])
            pltpu.sync_copy(a2_v, cell2_out.at[pl.ds(base, CHUNK)])

    plsc.subcore_barrier()
    for t in range(4):
        ch = sid + NS * t
        @pl.when(ch < N_CHUNKS)
        def _():
            sl = pl.ds(ch * CHUNK, CHUNK)
            @pl.when(cid == 0)
            def _():
                pltpu.sync_copy(accu_sp.at[sl], u0_out.at[sl])
                pltpu.sync_copy(accv_sp.at[sl], v0_out.at[sl])
                pltpu.sync_copy(accp_sp.at[sl], p0_out.at[sl])
                pltpu.sync_copy(accn_sp.at[sl], n0_out.at[sl])
            @pl.when(cid == 1)
            def _():
                pltpu.sync_copy(accu_sp.at[sl], u1_out.at[sl])
                pltpu.sync_copy(accv_sp.at[sl], v1_out.at[sl])
                pltpu.sync_copy(accp_sp.at[sl], p1_out.at[sl])
                pltpu.sync_copy(accn_sp.at[sl], n1_out.at[sl])


_N1 = jax.ShapeDtypeStruct((N_NODES,), jnp.float32)
_C1 = jax.ShapeDtypeStruct((N_CELLS,), jnp.float32)

_stage1 = pl.kernel(
    _stage1_body,
    out_type=(_N1, _N1, _N1, _N1, _N1, _N1, _N1, _N1, _C1, _C1, _C1),
    mesh=_mesh,
    compiler_params=_params,
    scratch_types=(
        [pltpu.VMEM_SHARED((N_NODES,), jnp.float32)] * 4
        + [pltpu.VMEM((CHUNK,), jnp.float32)] * 8
        + [pltpu.VMEM((CHUNK,), jnp.int32)] * 3
    ),
)


def _stage2_body(u0_hbm, v0_hbm, p0_hbm, n0_hbm, u1_hbm, v1_hbm, p1_hbm, n1_hbm,
                 nu_out, nv_out, np_out,
                 a_v, b_v, c0_v, c1_v, den_v, o_v):
    cid = lax.axis_index("c")
    sid = lax.axis_index("s")
    wid = sid * NC + cid

    for t in range(2):
        ch = wid + NW * t
        @pl.when(ch < N_CHUNKS)
        def _():
            sl = pl.ds(ch * CHUNK, CHUNK)
            pltpu.sync_copy(n0_hbm.at[sl], c0_v)
            pltpu.sync_copy(n1_hbm.at[sl], c1_v)

            def _den(i, c):
                s = pl.ds(i * LANES, LANES)
                den_v[s] = jnp.maximum(c0_v[s] + c1_v[s], 1.0)
                return c
            lax.fori_loop(0, GROUPS, _den, None)

            for (x0, x1, out) in ((u0_hbm, u1_hbm, nu_out),
                                  (v0_hbm, v1_hbm, nv_out),
                                  (p0_hbm, p1_hbm, np_out)):
                pltpu.sync_copy(x0.at[sl], a_v)
                pltpu.sync_copy(x1.at[sl], b_v)

                def _norm(i, c):
                    s = pl.ds(i * LANES, LANES)
                    o_v[s] = (a_v[s] + b_v[s]) / den_v[s]
                    return c
                lax.fori_loop(0, GROUPS, _norm, None)
                pltpu.sync_copy(o_v, out.at[sl])


_stage2 = pl.kernel(
    _stage2_body,
    out_type=(_N1, _N1, _N1),
    mesh=_mesh,
    compiler_params=_params,
    scratch_types=[pltpu.VMEM((CHUNK,), jnp.float32)] * 6,
)


@jax.jit
def kernel(predicted_edge_uvp, edge_index, face):
    u_col, v_col, p_col, senders, receivers = _split_edges(
        predicted_edge_uvp.T, edge_index)
    f0, f1, f2 = _split_faces(face)
    zeros = jnp.zeros((CHUNK,), jnp.float32)
    u0, v0, p0, n0, u1, v1, p1, n1, c0, c1, c2 = _stage1(
        u_col, v_col, p_col, senders, receivers, f0, f1, f2, zeros)
    nu, nv, np_ = _stage2(u0, v0, p0, n0, u1, v1, p1, n1)
    node_uvp = jnp.stack([nu, nv, np_], axis=1)
    cell_uvp = jnp.stack([c0, c1, c2], axis=1)
    return node_uvp, cell_uvp


# speedup vs baseline: 18.7366x; 1.1878x over previous
import jax
import jax.numpy as jnp
from jax import lax
from jax.experimental import pallas as pl
from jax.experimental.pallas import tpu as pltpu
from jax.experimental.pallas import tpu_sc as plsc

N_NODES = 100000
N_EDGES = 1600000
N_CELLS = 200000

NC = 2
NS = 16
NW = NC * NS

CHUNK = 2000
EPT = N_EDGES // NW
E_CHUNKS = EPT // CHUNK
C_CHUNKS = N_CELLS // CHUNK
N_CHUNKS = N_NODES // CHUNK
LANES = 16
GROUPS = CHUNK // LANES

COL_BLK = 12800

_mesh = plsc.VectorSubcoreMesh(core_axis_name="c", subcore_axis_name="s")
_params = pltpu.CompilerParams(use_tc_tiling_on_sc=False)


def _edge_body(ut_ref, ei_ref, u_ref, v_ref, p_ref, s_ref, r_ref):
    i = pl.program_id(0)
    sl = pl.ds(i * COL_BLK, COL_BLK)
    ut = ut_ref[...]
    u_ref[sl] = ut[0, :]
    v_ref[sl] = ut[1, :]
    p_ref[sl] = ut[2, :]
    ei = ei_ref[...]
    s_ref[sl] = ei[0, :]
    r_ref[sl] = ei[1, :]


_split_edges = pl.pallas_call(
    _edge_body,
    grid=(N_EDGES // COL_BLK,),
    in_specs=[pl.BlockSpec((3, COL_BLK), lambda i: (0, i)),
              pl.BlockSpec((2, COL_BLK), lambda i: (0, i))],
    out_specs=[pl.BlockSpec((N_EDGES,), lambda i: (0,))] * 5,
    out_shape=[jax.ShapeDtypeStruct((N_EDGES,), jnp.float32)] * 3
    + [jax.ShapeDtypeStruct((N_EDGES,), jnp.int32)] * 2,
)

def _face_body(f_ref, f0_ref, f1_ref, f2_ref):
    f = f_ref[...]
    f0_ref[...] = f[0, :]
    f1_ref[...] = f[1, :]
    f2_ref[...] = f[2, :]


_split_faces = pl.pallas_call(
    _face_body,
    out_shape=[jax.ShapeDtypeStruct((N_CELLS,), jnp.int32)] * 3,
)


def _stage1_body(u_hbm, v_hbm, p_hbm, send_hbm, recv_hbm,
                 f0_hbm, f1_hbm, f2_hbm, zeros_hbm,
                 u0_out, v0_out, p0_out, n0_out,
                 u1_out, v1_out, p1_out, n1_out,
                 cell0_out, cell1_out, cell2_out,
                 accu_sp, accv_sp, accp_sp, accn_sp,
                 su_v, sv_v, sp_v, ones_v, g_v, a0_v, a1_v, a2_v,
                 su2_v, sv2_v, sp2_v,
                 sidx_v, ridx_v, f_v, sidx2_v, ridx2_v, dma_sem):
    cid = lax.axis_index("c")
    sid = lax.axis_index("s")
    wid = sid * NC + cid

    for t in range(4):
        ch = sid + NS * t
        @pl.when(ch < N_CHUNKS)
        def _():
            sl = pl.ds(ch * CHUNK, CHUNK)
            pltpu.sync_copy(zeros_hbm, accu_sp.at[sl])
            pltpu.sync_copy(zeros_hbm, accv_sp.at[sl])
            pltpu.sync_copy(zeros_hbm, accp_sp.at[sl])
            pltpu.sync_copy(zeros_hbm, accn_sp.at[sl])

    def _fill_ones(i, carry):
        ones_v[pl.ds(i * LANES, LANES)] = jnp.full((LANES,), 1.0, jnp.float32)
        return carry
    lax.fori_loop(0, GROUPS, _fill_ones, None)

    plsc.subcore_barrier()

    bufs = ((su_v, sv_v, sp_v, sidx_v, ridx_v),
            (su2_v, sv2_v, sp2_v, sidx2_v, ridx2_v))

    def _load(j, bset):
        sl = pl.ds(wid * EPT + j * CHUNK, CHUNK)
        srcs = (u_hbm, v_hbm, p_hbm, send_hbm, recv_hbm)
        return [pltpu.async_copy(s.at[sl], b, dma_sem)
                for s, b in zip(srcs, bset)]

    cps = _load(0, bufs[0])
    for j in range(E_CHUNKS):
        for c in cps:
            c.wait()
        su, sv, sp_, sidx, ridx = bufs[j % 2]
        if j + 1 < E_CHUNKS:
            cps = _load(j + 1, bufs[(j + 1) % 2])
        for ix in (sidx, ridx):
            pltpu.sync_copy(su, accu_sp.at[ix], add=True)
            pltpu.sync_copy(sv, accv_sp.at[ix], add=True)
            pltpu.sync_copy(sp_, accp_sp.at[ix], add=True)
            pltpu.sync_copy(ones_v, accn_sp.at[ix], add=True)

    cols = (u_hbm, v_hbm, p_hbm)
    for t in range(4):
        ch = wid + NW * t
        @pl.when(ch < C_CHUNKS)
        def _():
            base = ch * CHUNK
            accs = (a0_v, a1_v, a2_v)
            for k, fk in enumerate((f0_hbm, f1_hbm, f2_hbm)):
                pltpu.sync_copy(fk.at[pl.ds(base, CHUNK)], f_v)
                for m in range(3):
                    if k == 0:
                        pltpu.sync_copy(cols[m].at[f_v], accs[m])
                    else:
                        pltpu.sync_copy(cols[m].at[f_v], g_v)

                        def _acc(i, c, am=accs[m]):
                            sl = pl.ds(i * LANES, LANES)
                            am[sl] = am[sl] + g_v[sl]
                            return c
                        lax.fori_loop(0, GROUPS, _acc, None)

            def _scale(i, c):
                sl = pl.ds(i * LANES, LANES)
                a0_v[sl] = a0_v[sl] / 3.0
                a1_v[sl] = a1_v[sl] / 3.0
                a2_v[sl] = a2_v[sl] / 3.0
                return c
            lax.fori_loop(0, GROUPS, _scale, None)

            pltpu.sync_copy(a0_v, cell0_out.at[pl.ds(base, CHUNK)])
            pltpu.sync_copy(a1_v, cell1_out.at[pl.ds(base, CHUNK)])
            pltpu.sync_copy(a2_v, cell2_out.at[pl.ds(base, CHUNK)])

    plsc.subcore_barrier()
    for t in range(4):
        ch = sid + NS * t
        @pl.when(ch < N_CHUNKS)
        def _():
            sl = pl.ds(ch * CHUNK, CHUNK)
            @pl.when(cid == 0)
            def _():
                pltpu.sync_copy(accu_sp.at[sl], u0_out.at[sl])
                pltpu.sync_copy(accv_sp.at[sl], v0_out.at[sl])
                pltpu.sync_copy(accp_sp.at[sl], p0_out.at[sl])
                pltpu.sync_copy(accn_sp.at[sl], n0_out.at[sl])
            @pl.when(cid == 1)
            def _():
                pltpu.sync_copy(accu_sp.at[sl], u1_out.at[sl])
                pltpu.sync_copy(accv_sp.at[sl], v1_out.at[sl])
                pltpu.sync_copy(accp_sp.at[sl], p1_out.at[sl])
                pltpu.sync_copy(accn_sp.at[sl], n1_out.at[sl])


_N1 = jax.ShapeDtypeStruct((N_NODES,), jnp.float32)
_C1 = jax.ShapeDtypeStruct((N_CELLS,), jnp.float32)

_stage1 = pl.kernel(
    _stage1_body,
    out_type=(_N1, _N1, _N1, _N1, _N1, _N1, _N1, _N1, _C1, _C1, _C1),
    mesh=_mesh,
    compiler_params=_params,
    scratch_types=(
        [pltpu.VMEM_SHARED((N_NODES,), jnp.float32)] * 4
        + [pltpu.VMEM((CHUNK,), jnp.float32)] * 11
        + [pltpu.VMEM((CHUNK,), jnp.int32)] * 5
        + [pltpu.SemaphoreType.DMA]
    ),
)


def _stage2_body(u0_hbm, v0_hbm, p0_hbm, n0_hbm, u1_hbm, v1_hbm, p1_hbm, n1_hbm,
                 nu_out, nv_out, np_out,
                 a_v, b_v, c0_v, c1_v, den_v, o_v):
    cid = lax.axis_index("c")
    sid = lax.axis_index("s")
    wid = sid * NC + cid

    for t in range(2):
        ch = wid + NW * t
        @pl.when(ch < N_CHUNKS)
        def _():
            sl = pl.ds(ch * CHUNK, CHUNK)
            pltpu.sync_copy(n0_hbm.at[sl], c0_v)
            pltpu.sync_copy(n1_hbm.at[sl], c1_v)

            def _den(i, c):
                s = pl.ds(i * LANES, LANES)
                den_v[s] = jnp.maximum(c0_v[s] + c1_v[s], 1.0)
                return c
            lax.fori_loop(0, GROUPS, _den, None)

            for (x0, x1, out) in ((u0_hbm, u1_hbm, nu_out),
                                  (v0_hbm, v1_hbm, nv_out),
                                  (p0_hbm, p1_hbm, np_out)):
                pltpu.sync_copy(x0.at[sl], a_v)
                pltpu.sync_copy(x1.at[sl], b_v)

                def _norm(i, c):
                    s = pl.ds(i * LANES, LANES)
                    o_v[s] = (a_v[s] + b_v[s]) / den_v[s]
                    return c
                lax.fori_loop(0, GROUPS, _norm, None)
                pltpu.sync_copy(o_v, out.at[sl])


_stage2 = pl.kernel(
    _stage2_body,
    out_type=(_N1, _N1, _N1),
    mesh=_mesh,
    compiler_params=_params,
    scratch_types=[pltpu.VMEM((CHUNK,), jnp.float32)] * 6,
)


@jax.jit
def kernel(predicted_edge_uvp, edge_index, face):
    u_col, v_col, p_col, senders, receivers = _split_edges(
        predicted_edge_uvp.T, edge_index)
    f0, f1, f2 = _split_faces(face)
    zeros = jnp.zeros((CHUNK,), jnp.float32)
    u0, v0, p0, n0, u1, v1, p1, n1, c0, c1, c2 = _stage1(
        u_col, v_col, p_col, senders, receivers, f0, f1, f2, zeros)
    nu, nv, np_ = _stage2(u0, v0, p0, n0, u1, v1, p1, n1)
    node_uvp = jnp.stack([nu, nv, np_], axis=1)
    cell_uvp = jnp.stack([c0, c1, c2], axis=1)
    return node_uvp, cell_uvp


# speedup vs baseline: 19.7539x; 1.0543x over previous
import jax
import jax.numpy as jnp
from jax import lax
from jax.experimental import pallas as pl
from jax.experimental.pallas import tpu as pltpu
from jax.experimental.pallas import tpu_sc as plsc

N_NODES = 100000
N_EDGES = 1600000
N_CELLS = 200000

NC = 2
NS = 16
NW = NC * NS

CHUNK = 2000
EPT = N_EDGES // NW
E_CHUNKS = EPT // CHUNK
C_CHUNKS = N_CELLS // CHUNK
N_CHUNKS = N_NODES // CHUNK
LANES = 16
GROUPS = CHUNK // LANES

COL_BLK = 12800

_mesh = plsc.VectorSubcoreMesh(core_axis_name="c", subcore_axis_name="s")
_params = pltpu.CompilerParams(use_tc_tiling_on_sc=False)


def _edge_body(ut_ref, ei_ref, u_ref, v_ref, p_ref, s_ref, r_ref):
    i = pl.program_id(0)
    sl = pl.ds(i * COL_BLK, COL_BLK)
    ut = ut_ref[...]
    u_ref[sl] = ut[0, :]
    v_ref[sl] = ut[1, :]
    p_ref[sl] = ut[2, :]
    ei = ei_ref[...]
    s_ref[sl] = ei[0, :]
    r_ref[sl] = ei[1, :]


_split_edges = pl.pallas_call(
    _edge_body,
    grid=(N_EDGES // COL_BLK,),
    in_specs=[pl.BlockSpec((3, COL_BLK), lambda i: (0, i)),
              pl.BlockSpec((2, COL_BLK), lambda i: (0, i))],
    out_specs=[pl.BlockSpec((N_EDGES,), lambda i: (0,))] * 5,
    out_shape=[jax.ShapeDtypeStruct((N_EDGES,), jnp.float32)] * 3
    + [jax.ShapeDtypeStruct((N_EDGES,), jnp.int32)] * 2,
)

def _face_body(f_ref, f0_ref, f1_ref, f2_ref):
    f = f_ref[...]
    f0_ref[...] = f[0, :]
    f1_ref[...] = f[1, :]
    f2_ref[...] = f[2, :]


_split_faces = pl.pallas_call(
    _face_body,
    out_shape=[jax.ShapeDtypeStruct((N_CELLS,), jnp.int32)] * 3,
)


def _stage1_body(u_hbm, v_hbm, p_hbm, send_hbm, recv_hbm,
                 f0_hbm, f1_hbm, f2_hbm, zeros_hbm,
                 u0_out, v0_out, p0_out, n0_out,
                 u1_out, v1_out, p1_out, n1_out,
                 cell0_out, cell1_out, cell2_out,
                 accu_sp, accv_sp, accp_sp, accn_sp,
                 su_v, sv_v, sp_v, ones_v, g_v, a0_v, a1_v, a2_v,
                 su2_v, sv2_v, sp2_v,
                 sidx_v, ridx_v, f_v, sidx2_v, ridx2_v, dma_sem):
    cid = lax.axis_index("c")
    sid = lax.axis_index("s")
    wid = sid * NC + cid

    for t in range(4):
        ch = sid + NS * t
        @pl.when(ch < N_CHUNKS)
        def _():
            sl = pl.ds(ch * CHUNK, CHUNK)
            pltpu.sync_copy(zeros_hbm, accu_sp.at[sl])
            pltpu.sync_copy(zeros_hbm, accv_sp.at[sl])
            pltpu.sync_copy(zeros_hbm, accp_sp.at[sl])
            pltpu.sync_copy(zeros_hbm, accn_sp.at[sl])

    def _fill_ones(i, carry):
        ones_v[pl.ds(i * LANES, LANES)] = jnp.full((LANES,), 1.0, jnp.float32)
        return carry
    lax.fori_loop(0, GROUPS, _fill_ones, None)

    plsc.subcore_barrier()

    bufs = ((su_v, sv_v, sp_v, sidx_v, ridx_v),
            (su2_v, sv2_v, sp2_v, sidx2_v, ridx2_v))

    def _load(j, bset):
        sl = pl.ds(wid * EPT + j * CHUNK, CHUNK)
        srcs = (u_hbm, v_hbm, p_hbm, send_hbm, recv_hbm)
        return [pltpu.async_copy(s.at[sl], b, dma_sem)
                for s, b in zip(srcs, bset)]

    cps = _load(0, bufs[0])
    for j in range(E_CHUNKS):
        for c in cps:
            c.wait()
        su, sv, sp_, sidx, ridx = bufs[j % 2]
        if j + 1 < E_CHUNKS:
            cps = _load(j + 1, bufs[(j + 1) % 2])
        for ix in (sidx, ridx):
            pltpu.sync_copy(su, accu_sp.at[ix], add=True)
            pltpu.sync_copy(sv, accv_sp.at[ix], add=True)
            pltpu.sync_copy(sp_, accp_sp.at[ix], add=True)
            pltpu.sync_copy(ones_v, accn_sp.at[ix], add=True)

    cols = (u_hbm, v_hbm, p_hbm)
    for t in range(4):
        ch = wid + NW * t
        @pl.when(ch < C_CHUNKS)
        def _():
            base = ch * CHUNK
            accs = (a0_v, a1_v, a2_v)
            gs = (g_v, su_v, sv_v)
            fs = (f_v, sidx_v, ridx_v)
            for c in [pltpu.async_copy(fk.at[pl.ds(base, CHUNK)], fb, dma_sem)
                      for fk, fb in zip((f0_hbm, f1_hbm, f2_hbm), fs)]:
                c.wait()
            for k in range(3):
                dsts = accs if k == 0 else gs
                for c in [pltpu.async_copy(cols[m].at[fs[k]], dsts[m], dma_sem)
                          for m in range(3)]:
                    c.wait()
                if k > 0:
                    for m in range(3):
                        def _acc(i, c, am=accs[m], gm=gs[m]):
                            sl = pl.ds(i * LANES, LANES)
                            am[sl] = am[sl] + gm[sl]
                            return c
                        lax.fori_loop(0, GROUPS, _acc, None)

            def _scale(i, c):
                sl = pl.ds(i * LANES, LANES)
                a0_v[sl] = a0_v[sl] / 3.0
                a1_v[sl] = a1_v[sl] / 3.0
                a2_v[sl] = a2_v[sl] / 3.0
                return c
            lax.fori_loop(0, GROUPS, _scale, None)

            pltpu.sync_copy(a0_v, cell0_out.at[pl.ds(base, CHUNK)])
            pltpu.sync_copy(a1_v, cell1_out.at[pl.ds(base, CHUNK)])
            pltpu.sync_copy(a2_v, cell2_out.at[pl.ds(base, CHUNK)])

    plsc.subcore_barrier()
    for t in range(4):
        ch = sid + NS * t
        @pl.when(ch < N_CHUNKS)
        def _():
            sl = pl.ds(ch * CHUNK, CHUNK)
            @pl.when(cid == 0)
            def _():
                pltpu.sync_copy(accu_sp.at[sl], u0_out.at[sl])
                pltpu.sync_copy(accv_sp.at[sl], v0_out.at[sl])
                pltpu.sync_copy(accp_sp.at[sl], p0_out.at[sl])
                pltpu.sync_copy(accn_sp.at[sl], n0_out.at[sl])
            @pl.when(cid == 1)
            def _():
                pltpu.sync_copy(accu_sp.at[sl], u1_out.at[sl])
                pltpu.sync_copy(accv_sp.at[sl], v1_out.at[sl])
                pltpu.sync_copy(accp_sp.at[sl], p1_out.at[sl])
                pltpu.sync_copy(accn_sp.at[sl], n1_out.at[sl])


_N1 = jax.ShapeDtypeStruct((N_NODES,), jnp.float32)
_C1 = jax.ShapeDtypeStruct((N_CELLS,), jnp.float32)

_stage1 = pl.kernel(
    _stage1_body,
    out_type=(_N1, _N1, _N1, _N1, _N1, _N1, _N1, _N1, _C1, _C1, _C1),
    mesh=_mesh,
    compiler_params=_params,
    scratch_types=(
        [pltpu.VMEM_SHARED((N_NODES,), jnp.float32)] * 4
        + [pltpu.VMEM((CHUNK,), jnp.float32)] * 11
        + [pltpu.VMEM((CHUNK,), jnp.int32)] * 5
        + [pltpu.SemaphoreType.DMA]
    ),
)


def _stage2_body(u0_hbm, v0_hbm, p0_hbm, n0_hbm, u1_hbm, v1_hbm, p1_hbm, n1_hbm,
                 nu_out, nv_out, np_out,
                 a_v, b_v, c0_v, c1_v, den_v, o_v):
    cid = lax.axis_index("c")
    sid = lax.axis_index("s")
    wid = sid * NC + cid

    for t in range(2):
        ch = wid + NW * t
        @pl.when(ch < N_CHUNKS)
        def _():
            sl = pl.ds(ch * CHUNK, CHUNK)
            pltpu.sync_copy(n0_hbm.at[sl], c0_v)
            pltpu.sync_copy(n1_hbm.at[sl], c1_v)

            def _den(i, c):
                s = pl.ds(i * LANES, LANES)
                den_v[s] = jnp.maximum(c0_v[s] + c1_v[s], 1.0)
                return c
            lax.fori_loop(0, GROUPS, _den, None)

            for (x0, x1, out) in ((u0_hbm, u1_hbm, nu_out),
                                  (v0_hbm, v1_hbm, nv_out),
                                  (p0_hbm, p1_hbm, np_out)):
                pltpu.sync_copy(x0.at[sl], a_v)
                pltpu.sync_copy(x1.at[sl], b_v)

                def _norm(i, c):
                    s = pl.ds(i * LANES, LANES)
                    o_v[s] = (a_v[s] + b_v[s]) / den_v[s]
                    return c
                lax.fori_loop(0, GROUPS, _norm, None)
                pltpu.sync_copy(o_v, out.at[sl])


_stage2 = pl.kernel(
    _stage2_body,
    out_type=(_N1, _N1, _N1),
    mesh=_mesh,
    compiler_params=_params,
    scratch_types=[pltpu.VMEM((CHUNK,), jnp.float32)] * 6,
)


@jax.jit
def kernel(predicted_edge_uvp, edge_index, face):
    u_col, v_col, p_col, senders, receivers = _split_edges(
        predicted_edge_uvp.T, edge_index)
    f0, f1, f2 = _split_faces(face)
    zeros = jnp.zeros((CHUNK,), jnp.float32)
    u0, v0, p0, n0, u1, v1, p1, n1, c0, c1, c2 = _stage1(
        u_col, v_col, p_col, senders, receivers, f0, f1, f2, zeros)
    nu, nv, np_ = _stage2(u0, v0, p0, n0, u1, v1, p1, n1)
    node_uvp = jnp.stack([nu, nv, np_], axis=1)
    cell_uvp = jnp.stack([c0, c1, c2], axis=1)
    return node_uvp, cell_uvp
